# Initial kernel scaffold; baseline (speedup 1.0000x reference)
import jax
import jax.numpy as jnp
from jax.experimental import pallas as pl


def kernel(observations, K1, b1, K2, b2, K3, b3, W_root, root_bias, attn_k, Wd, bd):
    raise NotImplementedError("write your pallas kernel here")


# speedup vs baseline: 2.7690x; 2.7690x over previous
import jax
import jax.numpy as jnp
from jax.experimental import pallas as pl

_N = 64
_F = 16
_S = 4
_C = 32
_KN = 64
_UNITS = 256


def _gnn_fused_kernel(x_ref, acol_ref, amat_ref, e_ref,
                      k1_ref, b1_ref, k2_ref, b2_ref, k3p_ref, b3r_ref,
                      wroot_ref, rootb_ref, attnk_ref, wd_ref, bd_ref,
                      out_ref):
    X = x_ref[0]
    Acol = acol_ref[0]
    Amat = amat_ref[0]
    E = e_ref[0]

    H1 = jnp.maximum(
        jnp.dot(E, k1_ref[...], preferred_element_type=jnp.float32)
        + b1_ref[...], 0.0)
    H2 = jnp.maximum(
        jnp.dot(H1, k2_ref[...], preferred_element_type=jnp.float32)
        + b2_ref[...], 0.0)

    mask = (Acol > 0.5).astype(jnp.float32)
    Hm3 = (H2 * mask).reshape(_N, _N, _KN)

    M3 = jnp.dot(X, k3p_ref[...],
                 preferred_element_type=jnp.float32).reshape(_N, _KN, _C)

    msum = jax.lax.dot_general(
        Hm3, M3, (((2,), (1,)), ((0,), (0,))),
        preferred_element_type=jnp.float32)
    agg = jnp.sum(msum, axis=0)

    maskmat = (Amat > 0.5).astype(jnp.float32)
    Xb3 = jnp.dot(X, b3r_ref[...],
                  preferred_element_type=jnp.float32)
    agg = agg + jnp.dot(maskmat, Xb3,
                        preferred_element_type=jnp.float32)

    Xc = jnp.maximum(
        agg + jnp.dot(X, wroot_ref[...],
                      preferred_element_type=jnp.float32)
        + rootb_ref[...], 0.0)

    lg = jnp.sum(Xc * attnk_ref[...], axis=1, keepdims=True)
    ex = jnp.exp(lg - jnp.max(lg))
    attn = ex / jnp.sum(ex)
    pooled = jnp.sum(attn * Xc, axis=0, keepdims=True)

    out_ref[0] = jnp.tanh(
        jnp.dot(pooled, wd_ref[...], preferred_element_type=jnp.float32)
        + bd_ref[...])


def kernel(observations, K1, b1, K2, b2, K3, b3, W_root, root_bias,
           attn_k, Wd, bd):
    Bc = observations.shape[0]
    NF, NN = _N * _F, _N * _N

    Xr = observations[:, :NF].reshape(Bc, _N, _F)
    Araw = observations[:, NF:NF + NN].reshape(Bc, _N, _N)
    E4 = observations[:, NF + NN:].reshape(Bc, _N, _N, _S)

    E4s = E4.transpose(0, 2, 1, 3).reshape(Bc, NN, _S)
    Acol = Araw.transpose(0, 2, 1).reshape(Bc, NN, 1)

    K3p = K3.reshape(_KN, _F, _C).transpose(1, 0, 2).reshape(_F, _KN * _C)
    b3r = b3.reshape(_F, _C)

    b1r = b1.reshape(1, _KN)
    b2r = b2.reshape(1, _KN)
    rootbr = root_bias.reshape(1, _C)
    attnkr = attn_k.reshape(1, _C)
    bdr = bd.reshape(1, _UNITS)

    def full(a):
        return pl.BlockSpec(a.shape, lambda b: (0,) * a.ndim)

    grid_spec = pl.GridSpec(
        grid=(Bc,),
        in_specs=[
            pl.BlockSpec((1, _N, _F), lambda b: (b, 0, 0)),
            pl.BlockSpec((1, NN, 1), lambda b: (b, 0, 0)),
            pl.BlockSpec((1, _N, _N), lambda b: (b, 0, 0)),
            pl.BlockSpec((1, NN, _S), lambda b: (b, 0, 0)),
            full(K1), full(b1r), full(K2), full(b2r), full(K3p), full(b3r),
            full(W_root), full(rootbr), full(attnkr), full(Wd), full(bdr),
        ],
        out_specs=pl.BlockSpec((1, 1, _UNITS), lambda b: (b, 0, 0)),
    )

    out = pl.pallas_call(
        _gnn_fused_kernel,
        grid_spec=grid_spec,
        out_shape=jax.ShapeDtypeStruct((Bc, 1, _UNITS), jnp.float32),
    )(Xr, Acol, Araw, E4s, K1, b1r, K2, b2r, K3p, b3r,
      W_root, rootbr, attnkr, Wd, bdr)
    return out.reshape(Bc, _UNITS)


# speedup vs baseline: 3.0170x; 1.0896x over previous
import jax
import jax.numpy as jnp
from jax.experimental import pallas as pl
from jax.experimental.pallas import tpu as pltpu

_N = 64
_F = 16
_S = 4
_C = 32
_KN = 64
_UNITS = 256
_G = 2


def _gnn_fused_kernel(x_ref, acol_ref, amat_ref, e_ref,
                      k1_ref, b1_ref, k2_ref, b2_ref, k3p_ref, b3r_ref,
                      wroot_ref, rootb_ref, attnk_ref, wd_ref, bd_ref,
                      out_ref):
    NN = _N * _N
    X = x_ref[...].reshape(_G * _N, _F)
    Acol = acol_ref[...].reshape(_G * NN, 1)
    E = e_ref[...].reshape(_G * NN, _S)

    H1 = jnp.maximum(
        jnp.dot(E, k1_ref[...], preferred_element_type=jnp.float32)
        + b1_ref[...], 0.0)
    H2 = jnp.maximum(
        jnp.dot(H1, k2_ref[...], preferred_element_type=jnp.float32)
        + b2_ref[...], 0.0)

    mask = (Acol > 0.5).astype(jnp.float32)
    Hm3 = (H2 * mask).reshape(_G * _N, _N, _KN)

    M3 = jnp.dot(X, k3p_ref[...],
                 preferred_element_type=jnp.float32).reshape(_G * _N, _KN, _C)

    msum = jax.lax.dot_general(
        Hm3, M3, (((2,), (1,)), ((0,), (0,))),
        preferred_element_type=jnp.float32)
    agg = jnp.sum(msum.reshape(_G, _N, _N, _C), axis=1)

    maskmat = (amat_ref[...] > 0.5).astype(jnp.float32)
    Xb3 = jnp.dot(X, b3r_ref[...],
                  preferred_element_type=jnp.float32).reshape(_G, _N, _C)
    agg = agg + jax.lax.dot_general(
        maskmat, Xb3, (((2,), (1,)), ((0,), (0,))),
        preferred_element_type=jnp.float32)

    XW = jnp.dot(X, wroot_ref[...],
                 preferred_element_type=jnp.float32).reshape(_G, _N, _C)
    Xc = jnp.maximum(agg + XW + rootb_ref[...], 0.0)

    lg = jnp.sum(Xc * attnk_ref[...], axis=2, keepdims=True)
    ex = jnp.exp(lg - jnp.max(lg, axis=1, keepdims=True))
    attn = ex / jnp.sum(ex, axis=1, keepdims=True)
    pooled = jnp.sum(attn * Xc, axis=1)

    out_ref[0] = jnp.tanh(
        jnp.dot(pooled, wd_ref[...], preferred_element_type=jnp.float32)
        + bd_ref[...])


def kernel(observations, K1, b1, K2, b2, K3, b3, W_root, root_bias,
           attn_k, Wd, bd):
    Bc = observations.shape[0]
    NF, NN = _N * _F, _N * _N

    Xr = observations[:, :NF].reshape(Bc, _N, _F)
    Araw = observations[:, NF:NF + NN].reshape(Bc, _N, _N)
    E4 = observations[:, NF + NN:].reshape(Bc, _N, _N, _S)

    E4s = E4.transpose(0, 2, 1, 3).reshape(Bc, NN, _S)
    Acol = Araw.transpose(0, 2, 1).reshape(Bc, NN, 1)

    K3p = K3.reshape(_KN, _F, _C).transpose(1, 0, 2).reshape(_F, _KN * _C)
    b3r = b3.reshape(_F, _C)

    b1r = b1.reshape(1, _KN)
    b2r = b2.reshape(1, _KN)
    rootbr = root_bias.reshape(1, _C)
    attnkr = attn_k.reshape(1, 1, _C)
    bdr = bd.reshape(1, _UNITS)

    def full(a):
        return pl.BlockSpec(a.shape, lambda b: (0,) * a.ndim)

    grid_spec = pl.GridSpec(
        grid=(Bc // _G,),
        in_specs=[
            pl.BlockSpec((_G, _N, _F), lambda b: (b, 0, 0)),
            pl.BlockSpec((_G, NN, 1), lambda b: (b, 0, 0)),
            pl.BlockSpec((_G, _N, _N), lambda b: (b, 0, 0)),
            pl.BlockSpec((_G, NN, _S), lambda b: (b, 0, 0)),
            full(K1), full(b1r), full(K2), full(b2r), full(K3p), full(b3r),
            full(W_root), full(rootbr), full(attnkr), full(Wd), full(bdr),
        ],
        out_specs=pl.BlockSpec((1, _G, _UNITS), lambda b: (b, 0, 0)),
    )

    out = pl.pallas_call(
        _gnn_fused_kernel,
        grid_spec=grid_spec,
        out_shape=jax.ShapeDtypeStruct((Bc // _G, _G, _UNITS), jnp.float32),
        compiler_params=pltpu.CompilerParams(
            dimension_semantics=("parallel",)),
    )(Xr, Acol, Araw, E4s, K1, b1r, K2, b2r, K3p, b3r,
      W_root, rootbr, attnkr, Wd, bdr)
    return out.reshape(Bc, _UNITS)


# speedup vs baseline: 4.9185x; 1.6303x over previous
import jax
import jax.numpy as jnp
from jax.experimental import pallas as pl
from jax.experimental.pallas import tpu as pltpu

_N = 64
_F = 16
_S = 4
_C = 32
_KN = 64
_UNITS = 256
_G = 2


def _gnn_fused_kernel(x_ref, amat_ref, e_ref,
                      k1_ref, b1_ref, k2_ref, b2_ref, k3p_ref, b3r_ref,
                      wroot_ref, rootb_ref, attnk_ref, wd_ref, bd_ref,
                      out_ref):
    NN = _N * _N
    X = x_ref[...].reshape(_G * _N, _F)
    E = e_ref[...].reshape(_G * NN, _S)
    maskmat = (amat_ref[...] > 0.5).astype(jnp.float32)

    H1 = jnp.maximum(
        jnp.dot(E, k1_ref[...], preferred_element_type=jnp.float32)
        + b1_ref[...], 0.0)
    H2 = jnp.maximum(
        jnp.dot(H1, k2_ref[...], preferred_element_type=jnp.float32)
        + b2_ref[...], 0.0)

    H3 = H2.reshape(_G * _N, _N, _KN)

    M3 = jnp.dot(X, k3p_ref[...],
                 preferred_element_type=jnp.float32).reshape(_G * _N, _KN, _C)

    msumT = jax.lax.dot_general(
        M3, H3, (((1,), (2,)), ((0,), (0,))),
        preferred_element_type=jnp.float32)
    msumT = msumT * maskmat.reshape(_G * _N, 1, _N)
    aggT = jnp.sum(msumT.reshape(_G, _N, _C, _N), axis=1)
    agg = jnp.transpose(aggT, (0, 2, 1))

    Xb3 = jnp.dot(X, b3r_ref[...],
                  preferred_element_type=jnp.float32).reshape(_G, _N, _C)
    agg = agg + jax.lax.dot_general(
        maskmat.reshape(_G, _N, _N), Xb3, (((1,), (1,)), ((0,), (0,))),
        preferred_element_type=jnp.float32)

    XW = jnp.dot(X, wroot_ref[...],
                 preferred_element_type=jnp.float32).reshape(_G, _N, _C)
    Xc = jnp.maximum(agg + XW + rootb_ref[...], 0.0)

    lg = jnp.sum(Xc * attnk_ref[...], axis=2, keepdims=True)
    ex = jnp.exp(lg - jnp.max(lg, axis=1, keepdims=True))
    attn = ex / jnp.sum(ex, axis=1, keepdims=True)
    pooled = jnp.sum(attn * Xc, axis=1)

    out_ref[0] = jnp.tanh(
        jnp.dot(pooled, wd_ref[...], preferred_element_type=jnp.float32)
        + bd_ref[...])


def kernel(observations, K1, b1, K2, b2, K3, b3, W_root, root_bias,
           attn_k, Wd, bd):
    Bc = observations.shape[0]
    NF, NN = _N * _F, _N * _N

    Xr = observations[:, :NF].reshape(Bc, _N, _F)
    Araw = observations[:, NF:NF + NN].reshape(Bc, _N, _N)
    E4 = observations[:, NF + NN:].reshape(Bc, _N, _N, _S)

    E4s = E4.transpose(0, 2, 1, 3).reshape(Bc, NN, _S)
    Amat_t = Araw.transpose(0, 2, 1)

    K3p = K3.reshape(_KN, _F, _C).transpose(1, 0, 2).reshape(_F, _KN * _C)
    b3r = b3.reshape(_F, _C)

    b1r = b1.reshape(1, _KN)
    b2r = b2.reshape(1, _KN)
    rootbr = root_bias.reshape(1, _C)
    attnkr = attn_k.reshape(1, 1, _C)
    bdr = bd.reshape(1, _UNITS)

    def full(a):
        return pl.BlockSpec(a.shape, lambda b: (0,) * a.ndim)

    grid_spec = pl.GridSpec(
        grid=(Bc // _G,),
        in_specs=[
            pl.BlockSpec((_G, _N, _F), lambda b: (b, 0, 0)),
            pl.BlockSpec((_G, _N, _N), lambda b: (b, 0, 0)),
            pl.BlockSpec((_G, NN, _S), lambda b: (b, 0, 0)),
            full(K1), full(b1r), full(K2), full(b2r), full(K3p), full(b3r),
            full(W_root), full(rootbr), full(attnkr), full(Wd), full(bdr),
        ],
        out_specs=pl.BlockSpec((1, _G, _UNITS), lambda b: (b, 0, 0)),
    )

    out = pl.pallas_call(
        _gnn_fused_kernel,
        grid_spec=grid_spec,
        out_shape=jax.ShapeDtypeStruct((Bc // _G, _G, _UNITS), jnp.float32),
        compiler_params=pltpu.CompilerParams(
            dimension_semantics=("parallel",)),
    )(Xr, Amat_t, E4s, K1, b1r, K2, b2r, K3p, b3r,
      W_root, rootbr, attnkr, Wd, bdr)
    return out.reshape(Bc, _UNITS)


# speedup vs baseline: 6.3696x; 1.2950x over previous
import jax
import jax.numpy as jnp
from jax.experimental import pallas as pl
from jax.experimental.pallas import tpu as pltpu

_N = 64
_F = 16
_S = 4
_C = 32
_KN = 64
_UNITS = 256
_G = 2


def _gnn_fused_kernel(x_ref, amat_ref, e_ref,
                      k1_ref, b1_ref, k2_ref, b2_ref, k3p_ref, b3r_ref,
                      wroot_ref, rootb_ref, attnk_ref, wd_ref, bd_ref,
                      out_ref):
    NN = _N * _N
    X = x_ref[...].reshape(_G * _N, _F)
    E = jnp.transpose(e_ref[...], (0, 2, 1)).reshape(_G * NN, _S)
    maskmat = (amat_ref[...] > 0.5).astype(jnp.float32)

    H1 = jnp.maximum(
        jnp.dot(E, k1_ref[...], preferred_element_type=jnp.float32)
        + b1_ref[...], 0.0)
    H2 = jnp.maximum(
        jnp.dot(H1, k2_ref[...], preferred_element_type=jnp.float32)
        + b2_ref[...], 0.0)

    H3 = H2.reshape(_G * _N, _N, _KN)

    M3 = jnp.dot(X, k3p_ref[...],
                 preferred_element_type=jnp.float32).reshape(_G * _N, _KN, _C)

    msumT = jax.lax.dot_general(
        M3, H3, (((1,), (2,)), ((0,), (0,))),
        preferred_element_type=jnp.float32)
    msumT = msumT * maskmat.reshape(_G * _N, 1, _N)
    aggT = jnp.sum(msumT.reshape(_G, _N, _C, _N), axis=1)
    agg = jnp.transpose(aggT, (0, 2, 1))

    Xb3 = jnp.dot(X, b3r_ref[...],
                  preferred_element_type=jnp.float32).reshape(_G, _N, _C)
    agg = agg + jax.lax.dot_general(
        maskmat.reshape(_G, _N, _N), Xb3, (((1,), (1,)), ((0,), (0,))),
        preferred_element_type=jnp.float32)

    XW = jnp.dot(X, wroot_ref[...],
                 preferred_element_type=jnp.float32).reshape(_G, _N, _C)
    Xc = jnp.maximum(agg + XW + rootb_ref[...], 0.0)

    lg = jnp.sum(Xc * attnk_ref[...], axis=2, keepdims=True)
    ex = jnp.exp(lg - jnp.max(lg, axis=1, keepdims=True))
    attn = ex / jnp.sum(ex, axis=1, keepdims=True)
    pooled = jnp.sum(attn * Xc, axis=1)

    out_ref[0] = jnp.tanh(
        jnp.dot(pooled, wd_ref[...], preferred_element_type=jnp.float32)
        + bd_ref[...])


def kernel(observations, K1, b1, K2, b2, K3, b3, W_root, root_bias,
           attn_k, Wd, bd):
    Bc = observations.shape[0]
    NF, NN = _N * _F, _N * _N

    Xr = observations[:, :NF].reshape(Bc, _N, _F)
    Araw = observations[:, NF:NF + NN].reshape(Bc, _N, _N)
    E4 = observations[:, NF + NN:].reshape(Bc, _N, _N, _S)

    E4s = E4.transpose(0, 3, 2, 1).reshape(Bc, _S, NN)
    Amat_t = Araw.transpose(0, 2, 1)

    K3p = K3.reshape(_KN, _F, _C).transpose(1, 0, 2).reshape(_F, _KN * _C)
    b3r = b3.reshape(_F, _C)

    b1r = b1.reshape(1, _KN)
    b2r = b2.reshape(1, _KN)
    rootbr = root_bias.reshape(1, _C)
    attnkr = attn_k.reshape(1, 1, _C)
    bdr = bd.reshape(1, _UNITS)

    def full(a):
        return pl.BlockSpec(a.shape, lambda b: (0,) * a.ndim)

    grid_spec = pl.GridSpec(
        grid=(Bc // _G,),
        in_specs=[
            pl.BlockSpec((_G, _N, _F), lambda b: (b, 0, 0)),
            pl.BlockSpec((_G, _N, _N), lambda b: (b, 0, 0)),
            pl.BlockSpec((_G, _S, NN), lambda b: (b, 0, 0)),
            full(K1), full(b1r), full(K2), full(b2r), full(K3p), full(b3r),
            full(W_root), full(rootbr), full(attnkr), full(Wd), full(bdr),
        ],
        out_specs=pl.BlockSpec((1, _G, _UNITS), lambda b: (b, 0, 0)),
    )

    out = pl.pallas_call(
        _gnn_fused_kernel,
        grid_spec=grid_spec,
        out_shape=jax.ShapeDtypeStruct((Bc // _G, _G, _UNITS), jnp.float32),
        compiler_params=pltpu.CompilerParams(
            dimension_semantics=("parallel",)),
    )(Xr, Amat_t, E4s, K1, b1r, K2, b2r, K3p, b3r,
      W_root, rootbr, attnkr, Wd, bdr)
    return out.reshape(Bc, _UNITS)


# speedup vs baseline: 6.4782x; 1.0171x over previous
import jax
import jax.numpy as jnp
from jax.experimental import pallas as pl
from jax.experimental.pallas import tpu as pltpu

_N = 64
_F = 16
_S = 4
_C = 32
_KN = 64
_UNITS = 256
_G = 4


def _gnn_fused_kernel(x_ref, amat_ref, e_ref,
                      k1_ref, b1_ref, k2_ref, b2_ref, k3p_ref, b3r_ref,
                      wroot_ref, rootb_ref, attnk_ref, wd_ref, bd_ref,
                      out_ref):
    NN = _N * _N
    X = x_ref[...].reshape(_G * _N, _F)
    E = jnp.transpose(e_ref[...], (0, 2, 1)).reshape(_G * NN, _S)
    maskmat = (amat_ref[...] > 0.5).astype(jnp.float32)

    H1 = jnp.maximum(
        jnp.dot(E, k1_ref[...], preferred_element_type=jnp.float32)
        + b1_ref[...], 0.0)
    H2 = jnp.maximum(
        jnp.dot(H1, k2_ref[...], preferred_element_type=jnp.float32)
        + b2_ref[...], 0.0)

    H3 = H2.reshape(_G * _N, _N, _KN)

    M3 = jnp.dot(X, k3p_ref[...],
                 preferred_element_type=jnp.float32).reshape(_G * _N, _KN, _C)

    msumT = jax.lax.dot_general(
        M3, H3, (((1,), (2,)), ((0,), (0,))),
        preferred_element_type=jnp.float32)
    msumT = msumT * maskmat.reshape(_G * _N, 1, _N)
    aggT = jnp.sum(msumT.reshape(_G, _N, _C, _N), axis=1)
    agg = jnp.transpose(aggT, (0, 2, 1))

    Xb3 = jnp.dot(X, b3r_ref[...],
                  preferred_element_type=jnp.float32).reshape(_G, _N, _C)
    agg = agg + jax.lax.dot_general(
        maskmat.reshape(_G, _N, _N), Xb3, (((1,), (1,)), ((0,), (0,))),
        preferred_element_type=jnp.float32)

    XW = jnp.dot(X, wroot_ref[...],
                 preferred_element_type=jnp.float32).reshape(_G, _N, _C)
    Xc = jnp.maximum(agg + XW + rootb_ref[...], 0.0)

    lg = jnp.sum(Xc * attnk_ref[...], axis=2, keepdims=True)
    ex = jnp.exp(lg - jnp.max(lg, axis=1, keepdims=True))
    attn = ex / jnp.sum(ex, axis=1, keepdims=True)
    pooled = jnp.sum(attn * Xc, axis=1)

    out_ref[0] = jnp.tanh(
        jnp.dot(pooled, wd_ref[...], preferred_element_type=jnp.float32)
        + bd_ref[...])


def kernel(observations, K1, b1, K2, b2, K3, b3, W_root, root_bias,
           attn_k, Wd, bd):
    Bc = observations.shape[0]
    NF, NN = _N * _F, _N * _N

    Xr = observations[:, :NF].reshape(Bc, _N, _F)
    Araw = observations[:, NF:NF + NN].reshape(Bc, _N, _N)
    E4 = observations[:, NF + NN:].reshape(Bc, _N, _N, _S)

    E4s = E4.transpose(0, 3, 2, 1).reshape(Bc, _S, NN)
    Amat_t = Araw.transpose(0, 2, 1)

    K3p = K3.reshape(_KN, _F, _C).transpose(1, 0, 2).reshape(_F, _KN * _C)
    b3r = b3.reshape(_F, _C)

    b1r = b1.reshape(1, _KN)
    b2r = b2.reshape(1, _KN)
    rootbr = root_bias.reshape(1, _C)
    attnkr = attn_k.reshape(1, 1, _C)
    bdr = bd.reshape(1, _UNITS)

    def full(a):
        return pl.BlockSpec(a.shape, lambda b: (0,) * a.ndim)

    grid_spec = pl.GridSpec(
        grid=(Bc // _G,),
        in_specs=[
            pl.BlockSpec((_G, _N, _F), lambda b: (b, 0, 0)),
            pl.BlockSpec((_G, _N, _N), lambda b: (b, 0, 0)),
            pl.BlockSpec((_G, _S, NN), lambda b: (b, 0, 0)),
            full(K1), full(b1r), full(K2), full(b2r), full(K3p), full(b3r),
            full(W_root), full(rootbr), full(attnkr), full(Wd), full(bdr),
        ],
        out_specs=pl.BlockSpec((1, _G, _UNITS), lambda b: (b, 0, 0)),
    )

    out = pl.pallas_call(
        _gnn_fused_kernel,
        grid_spec=grid_spec,
        out_shape=jax.ShapeDtypeStruct((Bc // _G, _G, _UNITS), jnp.float32),
        compiler_params=pltpu.CompilerParams(
            dimension_semantics=("parallel",)),
    )(Xr, Amat_t, E4s, K1, b1r, K2, b2r, K3p, b3r,
      W_root, rootbr, attnkr, Wd, bdr)
    return out.reshape(Bc, _UNITS)


# speedup vs baseline: 7.4522x; 1.1503x over previous
import jax
import jax.numpy as jnp
from jax.experimental import pallas as pl
from jax.experimental.pallas import tpu as pltpu

_N = 64
_F = 16
_S = 4
_C = 32
_KN = 64
_UNITS = 256
_G = 4


def _gnn_fused_kernel(x_ref, amat_ref, e_ref,
                      k1_ref, b1_ref, k2_ref, b2_ref, k3p_ref, b3r_ref,
                      wroot_ref, rootb_ref, attnk_ref, wd_ref, bd_ref,
                      out_ref):
    NN = _N * _N
    X = x_ref[...].reshape(_G * _N, _F)
    et = jnp.transpose(e_ref[...], (0, 2, 1))
    et = jnp.transpose(et.reshape(_G, _N, _S, _N), (0, 1, 3, 2))
    E = et.reshape(_G * NN, _S)
    amat_t = jnp.transpose(amat_ref[...], (0, 2, 1))
    maskmat = (amat_t > 0.5).astype(jnp.float32)

    H1 = jnp.maximum(
        jnp.dot(E, k1_ref[...], preferred_element_type=jnp.float32)
        + b1_ref[...], 0.0)
    H2 = jnp.maximum(
        jnp.dot(H1, k2_ref[...], preferred_element_type=jnp.float32)
        + b2_ref[...], 0.0)

    H3 = H2.reshape(_G * _N, _N, _KN)

    M3 = jnp.dot(X, k3p_ref[...],
                 preferred_element_type=jnp.float32).reshape(_G * _N, _KN, _C)

    msumT = jax.lax.dot_general(
        M3, H3, (((1,), (2,)), ((0,), (0,))),
        preferred_element_type=jnp.float32)
    msumT = msumT * maskmat.reshape(_G * _N, 1, _N)
    aggT = jnp.sum(msumT.reshape(_G, _N, _C, _N), axis=1)
    agg = jnp.transpose(aggT, (0, 2, 1))

    Xb3 = jnp.dot(X, b3r_ref[...],
                  preferred_element_type=jnp.float32).reshape(_G, _N, _C)
    agg = agg + jax.lax.dot_general(
        maskmat.reshape(_G, _N, _N), Xb3, (((1,), (1,)), ((0,), (0,))),
        preferred_element_type=jnp.float32)

    XW = jnp.dot(X, wroot_ref[...],
                 preferred_element_type=jnp.float32).reshape(_G, _N, _C)
    Xc = jnp.maximum(agg + XW + rootb_ref[...], 0.0)

    lg = jnp.sum(Xc * attnk_ref[...], axis=2, keepdims=True)
    ex = jnp.exp(lg - jnp.max(lg, axis=1, keepdims=True))
    attn = ex / jnp.sum(ex, axis=1, keepdims=True)
    pooled = jnp.sum(attn * Xc, axis=1)

    out_ref[0] = jnp.tanh(
        jnp.dot(pooled, wd_ref[...], preferred_element_type=jnp.float32)
        + bd_ref[...])


def kernel(observations, K1, b1, K2, b2, K3, b3, W_root, root_bias,
           attn_k, Wd, bd):
    Bc = observations.shape[0]
    NF, NN = _N * _F, _N * _N

    Xr = observations[:, :NF].reshape(Bc, _N, _F)
    Araw = observations[:, NF:NF + NN].reshape(Bc, _N, _N)
    E4s = observations[:, NF + NN:].reshape(Bc, _N, _N * _S)

    K3p = K3.reshape(_KN, _F, _C).transpose(1, 0, 2).reshape(_F, _KN * _C)
    b3r = b3.reshape(_F, _C)

    b1r = b1.reshape(1, _KN)
    b2r = b2.reshape(1, _KN)
    rootbr = root_bias.reshape(1, _C)
    attnkr = attn_k.reshape(1, 1, _C)
    bdr = bd.reshape(1, _UNITS)

    def full(a):
        return pl.BlockSpec(a.shape, lambda b: (0,) * a.ndim)

    grid_spec = pl.GridSpec(
        grid=(Bc // _G,),
        in_specs=[
            pl.BlockSpec((_G, _N, _F), lambda b: (b, 0, 0)),
            pl.BlockSpec((_G, _N, _N), lambda b: (b, 0, 0)),
            pl.BlockSpec((_G, _N, _N * _S), lambda b: (b, 0, 0)),
            full(K1), full(b1r), full(K2), full(b2r), full(K3p), full(b3r),
            full(W_root), full(rootbr), full(attnkr), full(Wd), full(bdr),
        ],
        out_specs=pl.BlockSpec((1, _G, _UNITS), lambda b: (b, 0, 0)),
    )

    out = pl.pallas_call(
        _gnn_fused_kernel,
        grid_spec=grid_spec,
        out_shape=jax.ShapeDtypeStruct((Bc // _G, _G, _UNITS), jnp.float32),
        compiler_params=pltpu.CompilerParams(
            dimension_semantics=("parallel",)),
    )(Xr, Araw, E4s, K1, b1r, K2, b2r, K3p, b3r,
      W_root, rootbr, attnkr, Wd, bdr)
    return out.reshape(Bc, _UNITS)


# speedup vs baseline: 8.9405x; 1.1997x over previous
import jax
import jax.numpy as jnp
from jax.experimental import pallas as pl
from jax.experimental.pallas import tpu as pltpu

_N = 64
_F = 16
_S = 4
_C = 32
_KN = 64
_UNITS = 256
_G = 4


def _gnn_fused_kernel(x_ref, amat_ref, e_ref,
                      k1_ref, b1_ref, k2_ref, b2_ref, k3p_ref, b3r_ref,
                      wroot_ref, rootb_ref, attnk_ref, wd_ref, bd_ref,
                      out_ref):
    NN = _N * _N
    X = x_ref[...].reshape(_G * _N, _F)
    et = jnp.transpose(e_ref[...].astype(jnp.bfloat16), (0, 2, 1))
    E_jsi = et.reshape(_G * _N, _S, _N)
    amat_t = jnp.transpose(amat_ref[...], (0, 2, 1))
    maskmat = (amat_t > 0.5).astype(jnp.float32)

    H1 = jnp.maximum(
        jax.lax.dot_general(E_jsi, k1_ref[...], (((1,), (0,)), ((), ())),
                            preferred_element_type=jnp.float32)
        + b1_ref[...], 0.0).astype(jnp.bfloat16)
    H3 = jnp.maximum(
        jax.lax.dot_general(H1, k2_ref[...], (((2,), (0,)), ((), ())),
                            preferred_element_type=jnp.float32)
        + b2_ref[...], 0.0).astype(jnp.bfloat16)

    M3 = jnp.dot(X.astype(jnp.bfloat16), k3p_ref[...],
                 preferred_element_type=jnp.float32)
    M3 = M3.astype(jnp.bfloat16).reshape(_G * _N, _KN, _C)

    msumT = jax.lax.dot_general(
        M3, H3, (((1,), (2,)), ((0,), (0,))),
        preferred_element_type=jnp.float32)
    msumT = msumT * maskmat.reshape(_G * _N, 1, _N)
    aggT = jnp.sum(msumT.reshape(_G, _N, _C, _N), axis=1)
    agg = jnp.transpose(aggT, (0, 2, 1))

    Xb3 = jnp.dot(X, b3r_ref[...],
                  preferred_element_type=jnp.float32).reshape(_G, _N, _C)
    agg = agg + jax.lax.dot_general(
        maskmat.reshape(_G, _N, _N), Xb3, (((1,), (1,)), ((0,), (0,))),
        preferred_element_type=jnp.float32)

    XW = jnp.dot(X, wroot_ref[...],
                 preferred_element_type=jnp.float32).reshape(_G, _N, _C)
    Xc = jnp.maximum(agg + XW + rootb_ref[...], 0.0)

    lg = jnp.sum(Xc * attnk_ref[...], axis=2, keepdims=True)
    ex = jnp.exp(lg - jnp.max(lg, axis=1, keepdims=True))
    attn = ex / jnp.sum(ex, axis=1, keepdims=True)
    pooled = jnp.sum(attn * Xc, axis=1)

    out_ref[0] = jnp.tanh(
        jnp.dot(pooled, wd_ref[...], preferred_element_type=jnp.float32)
        + bd_ref[...])


def kernel(observations, K1, b1, K2, b2, K3, b3, W_root, root_bias,
           attn_k, Wd, bd):
    Bc = observations.shape[0]
    NF, NN = _N * _F, _N * _N

    Xr = observations[:, :NF].reshape(Bc, _N, _F)
    Araw = observations[:, NF:NF + NN].reshape(Bc, _N, _N)
    E4s = observations[:, NF + NN:].reshape(Bc, _N, _N * _S)

    K3p = K3.reshape(_KN, _F, _C).transpose(1, 0, 2).reshape(_F, _KN * _C)
    K1b = K1.astype(jnp.bfloat16)
    K2b = K2.astype(jnp.bfloat16)
    K3pb = K3p.astype(jnp.bfloat16)
    b3r = b3.reshape(_F, _C)

    b1r = b1.reshape(1, _KN)
    b2r = b2.reshape(1, _KN)
    rootbr = root_bias.reshape(1, _C)
    attnkr = attn_k.reshape(1, 1, _C)
    bdr = bd.reshape(1, _UNITS)

    def full(a):
        return pl.BlockSpec(a.shape, lambda b: (0,) * a.ndim)

    grid_spec = pl.GridSpec(
        grid=(Bc // _G,),
        in_specs=[
            pl.BlockSpec((_G, _N, _F), lambda b: (b, 0, 0)),
            pl.BlockSpec((_G, _N, _N), lambda b: (b, 0, 0)),
            pl.BlockSpec((_G, _N, _N * _S), lambda b: (b, 0, 0)),
            full(K1b), full(b1r), full(K2b), full(b2r), full(K3pb), full(b3r),
            full(W_root), full(rootbr), full(attnkr), full(Wd), full(bdr),
        ],
        out_specs=pl.BlockSpec((1, _G, _UNITS), lambda b: (b, 0, 0)),
    )

    out = pl.pallas_call(
        _gnn_fused_kernel,
        grid_spec=grid_spec,
        out_shape=jax.ShapeDtypeStruct((Bc // _G, _G, _UNITS), jnp.float32),
        compiler_params=pltpu.CompilerParams(
            dimension_semantics=("parallel",)),
    )(Xr, Araw, E4s, K1b, b1r, K2b, b2r, K3pb, b3r,
      W_root, rootbr, attnkr, Wd, bdr)
    return out.reshape(Bc, _UNITS)


# speedup vs baseline: 9.4134x; 1.0529x over previous
import jax
import jax.numpy as jnp
from jax.experimental import pallas as pl
from jax.experimental.pallas import tpu as pltpu

_N = 64
_F = 16
_S = 4
_C = 32
_KN = 64
_UNITS = 256
_G = 8


def _gnn_fused_kernel(x_ref, amat_ref, e_ref,
                      k1_ref, b1_ref, k2_ref, b2_ref, k3p_ref, b3r_ref,
                      wroot_ref, rootb_ref, attnk_ref, wd_ref, bd_ref,
                      out_ref):
    NN = _N * _N
    X = x_ref[...].reshape(_G * _N, _F)
    et = jnp.transpose(e_ref[...].astype(jnp.bfloat16), (0, 2, 1))
    E_jsi = et.reshape(_G * _N, _S, _N)
    amat_t = jnp.transpose(amat_ref[...], (0, 2, 1))
    maskmat = (amat_t > 0.5).astype(jnp.float32)

    H1 = jnp.maximum(
        jax.lax.dot_general(E_jsi, k1_ref[...], (((1,), (0,)), ((), ())),
                            preferred_element_type=jnp.float32)
        + b1_ref[...], 0.0).astype(jnp.bfloat16)
    H3 = jnp.maximum(
        jax.lax.dot_general(H1, k2_ref[...], (((2,), (0,)), ((), ())),
                            preferred_element_type=jnp.float32)
        + b2_ref[...], 0.0).astype(jnp.bfloat16)

    M3 = jnp.dot(X.astype(jnp.bfloat16), k3p_ref[...],
                 preferred_element_type=jnp.float32)
    M3 = M3.astype(jnp.bfloat16).reshape(_G * _N, _KN, _C)

    msumT = jax.lax.dot_general(
        M3, H3, (((1,), (2,)), ((0,), (0,))),
        preferred_element_type=jnp.float32)
    msumT = msumT * maskmat.reshape(_G * _N, 1, _N)
    aggT = jnp.sum(msumT.reshape(_G, _N, _C, _N), axis=1)
    agg = jnp.transpose(aggT, (0, 2, 1))

    Xb3 = jnp.dot(X, b3r_ref[...],
                  preferred_element_type=jnp.float32).reshape(_G, _N, _C)
    agg = agg + jax.lax.dot_general(
        maskmat.reshape(_G, _N, _N), Xb3, (((1,), (1,)), ((0,), (0,))),
        preferred_element_type=jnp.float32)

    XW = jnp.dot(X, wroot_ref[...],
                 preferred_element_type=jnp.float32).reshape(_G, _N, _C)
    Xc = jnp.maximum(agg + XW + rootb_ref[...], 0.0)

    lg = jnp.sum(Xc * attnk_ref[...], axis=2, keepdims=True)
    ex = jnp.exp(lg - jnp.max(lg, axis=1, keepdims=True))
    attn = ex / jnp.sum(ex, axis=1, keepdims=True)
    pooled = jnp.sum(attn * Xc, axis=1)

    out_ref[0] = jnp.tanh(
        jnp.dot(pooled, wd_ref[...], preferred_element_type=jnp.float32)
        + bd_ref[...])


def kernel(observations, K1, b1, K2, b2, K3, b3, W_root, root_bias,
           attn_k, Wd, bd):
    Bc = observations.shape[0]
    NF, NN = _N * _F, _N * _N

    Xr = observations[:, :NF].reshape(Bc, _N, _F)
    Araw = observations[:, NF:NF + NN].reshape(Bc, _N, _N)
    E4s = observations[:, NF + NN:].reshape(Bc, _N, _N * _S)

    K3p = K3.reshape(_KN, _F, _C).transpose(1, 0, 2).reshape(_F, _KN * _C)
    K1b = K1.astype(jnp.bfloat16)
    K2b = K2.astype(jnp.bfloat16)
    K3pb = K3p.astype(jnp.bfloat16)
    b3r = b3.reshape(_F, _C)

    b1r = b1.reshape(1, _KN)
    b2r = b2.reshape(1, _KN)
    rootbr = root_bias.reshape(1, _C)
    attnkr = attn_k.reshape(1, 1, _C)
    bdr = bd.reshape(1, _UNITS)

    def full(a):
        return pl.BlockSpec(a.shape, lambda b: (0,) * a.ndim)

    grid_spec = pl.GridSpec(
        grid=(Bc // _G,),
        in_specs=[
            pl.BlockSpec((_G, _N, _F), lambda b: (b, 0, 0)),
            pl.BlockSpec((_G, _N, _N), lambda b: (b, 0, 0)),
            pl.BlockSpec((_G, _N, _N * _S), lambda b: (b, 0, 0)),
            full(K1b), full(b1r), full(K2b), full(b2r), full(K3pb), full(b3r),
            full(W_root), full(rootbr), full(attnkr), full(Wd), full(bdr),
        ],
        out_specs=pl.BlockSpec((1, _G, _UNITS), lambda b: (b, 0, 0)),
    )

    out = pl.pallas_call(
        _gnn_fused_kernel,
        grid_spec=grid_spec,
        out_shape=jax.ShapeDtypeStruct((Bc // _G, _G, _UNITS), jnp.float32),
        compiler_params=pltpu.CompilerParams(
            dimension_semantics=("parallel",)),
    )(Xr, Araw, E4s, K1b, b1r, K2b, b2r, K3pb, b3r,
      W_root, rootbr, attnkr, Wd, bdr)
    return out.reshape(Bc, _UNITS)


# speedup vs baseline: 9.6662x; 1.0269x over previous
import jax
import jax.numpy as jnp
from jax.experimental import pallas as pl
from jax.experimental.pallas import tpu as pltpu

_N = 64
_F = 16
_S = 4
_C = 32
_KN = 64
_UNITS = 256
_G = 16


def _gnn_fused_kernel(x_ref, amat_ref, e_ref,
                      k1_ref, b1_ref, k2_ref, b2_ref, k3p_ref, b3r_ref,
                      wroot_ref, rootb_ref, attnk_ref, wd_ref, bd_ref,
                      out_ref):
    NN = _N * _N
    X = x_ref[...].reshape(_G * _N, _F)
    et = jnp.transpose(e_ref[...].astype(jnp.bfloat16), (0, 2, 1))
    E_jsi = et.reshape(_G * _N, _S, _N)
    amat_t = jnp.transpose(amat_ref[...], (0, 2, 1))
    maskmat = (amat_t > 0.5).astype(jnp.float32)

    H1 = jnp.maximum(
        jax.lax.dot_general(E_jsi, k1_ref[...], (((1,), (0,)), ((), ())),
                            preferred_element_type=jnp.float32)
        + b1_ref[...], 0.0).astype(jnp.bfloat16)
    H3 = jnp.maximum(
        jax.lax.dot_general(H1, k2_ref[...], (((2,), (0,)), ((), ())),
                            preferred_element_type=jnp.float32)
        + b2_ref[...], 0.0).astype(jnp.bfloat16)

    M3 = jnp.dot(X.astype(jnp.bfloat16), k3p_ref[...],
                 preferred_element_type=jnp.float32)
    M3 = M3.astype(jnp.bfloat16).reshape(_G * _N, _KN, _C)

    msumT = jax.lax.dot_general(
        M3, H3, (((1,), (2,)), ((0,), (0,))),
        preferred_element_type=jnp.float32)
    msumT = msumT * maskmat.reshape(_G * _N, 1, _N)
    aggT = jnp.sum(msumT.reshape(_G, _N, _C, _N), axis=1)
    agg = jnp.transpose(aggT, (0, 2, 1))

    Xb3 = jnp.dot(X, b3r_ref[...],
                  preferred_element_type=jnp.float32).reshape(_G, _N, _C)
    agg = agg + jax.lax.dot_general(
        maskmat.reshape(_G, _N, _N), Xb3, (((1,), (1,)), ((0,), (0,))),
        preferred_element_type=jnp.float32)

    XW = jnp.dot(X, wroot_ref[...],
                 preferred_element_type=jnp.float32).reshape(_G, _N, _C)
    Xc = jnp.maximum(agg + XW + rootb_ref[...], 0.0)

    lg = jnp.sum(Xc * attnk_ref[...], axis=2, keepdims=True)
    ex = jnp.exp(lg - jnp.max(lg, axis=1, keepdims=True))
    attn = ex / jnp.sum(ex, axis=1, keepdims=True)
    pooled = jnp.sum(attn * Xc, axis=1)

    out_ref[0] = jnp.tanh(
        jnp.dot(pooled, wd_ref[...], preferred_element_type=jnp.float32)
        + bd_ref[...])


def kernel(observations, K1, b1, K2, b2, K3, b3, W_root, root_bias,
           attn_k, Wd, bd):
    Bc = observations.shape[0]
    NF, NN = _N * _F, _N * _N

    Xr = observations[:, :NF].reshape(Bc, _N, _F)
    Araw = observations[:, NF:NF + NN].reshape(Bc, _N, _N)
    E4s = observations[:, NF + NN:].reshape(Bc, _N, _N * _S)

    K3p = K3.reshape(_KN, _F, _C).transpose(1, 0, 2).reshape(_F, _KN * _C)
    K1b = K1.astype(jnp.bfloat16)
    K2b = K2.astype(jnp.bfloat16)
    K3pb = K3p.astype(jnp.bfloat16)
    b3r = b3.reshape(_F, _C)

    b1r = b1.reshape(1, _KN)
    b2r = b2.reshape(1, _KN)
    rootbr = root_bias.reshape(1, _C)
    attnkr = attn_k.reshape(1, 1, _C)
    bdr = bd.reshape(1, _UNITS)

    def full(a):
        return pl.BlockSpec(a.shape, lambda b: (0,) * a.ndim)

    grid_spec = pl.GridSpec(
        grid=(Bc // _G,),
        in_specs=[
            pl.BlockSpec((_G, _N, _F), lambda b: (b, 0, 0)),
            pl.BlockSpec((_G, _N, _N), lambda b: (b, 0, 0)),
            pl.BlockSpec((_G, _N, _N * _S), lambda b: (b, 0, 0)),
            full(K1b), full(b1r), full(K2b), full(b2r), full(K3pb), full(b3r),
            full(W_root), full(rootbr), full(attnkr), full(Wd), full(bdr),
        ],
        out_specs=pl.BlockSpec((1, _G, _UNITS), lambda b: (b, 0, 0)),
    )

    out = pl.pallas_call(
        _gnn_fused_kernel,
        grid_spec=grid_spec,
        out_shape=jax.ShapeDtypeStruct((Bc // _G, _G, _UNITS), jnp.float32),
        compiler_params=pltpu.CompilerParams(
            dimension_semantics=("parallel",)),
    )(Xr, Araw, E4s, K1b, b1r, K2b, b2r, K3pb, b3r,
      W_root, rootbr, attnkr, Wd, bdr)
    return out.reshape(Bc, _UNITS)
